# Initial kernel scaffold; baseline (speedup 1.0000x reference)
import jax
import jax.numpy as jnp
from jax import lax
from jax.experimental import pallas as pl
from jax.experimental.pallas import tpu as pltpu
from jax.experimental.pallas import tpu_sc as plsc

B = 16384
L = 20
D = 32
DD = 2 * D
NC = 2
NS = 16
NW = NC * NS
BPW = B // NW
CH = 64
NCHUNK = BPW // CH
GSZ = 128
NG_T = (CH * L) // GSZ
NG_I = BPW // GSZ


def _treesum(vs):
    vs = list(vs)
    while len(vs) > 1:
        nxt = [vs[i] + vs[i + 1] for i in range(0, len(vs) - 1, 2)]
        if len(vs) % 2:
            nxt.append(vs[-1])
        vs = nxt
    return vs[0]


def _body(ids_hbm, toks_hbm, idtab_hbm, titab_hbm, out_hbm,
          tok_v, ids_v, idrows_v, grows_v, out_v, row0_v, nz_v, inv_v,
          sem_id, sem_g0, sem_g1, sem_o0, sem_o1):
    wid = lax.axis_index("s") * NC + lax.axis_index("c")
    base = wid * BPW

    pltpu.sync_copy(toks_hbm.at[pl.ds(base * L, BPW * L)], tok_v)
    pltpu.sync_copy(ids_hbm.at[pl.ds(base, BPW)], ids_v)
    pltpu.sync_copy(titab_hbm.at[pl.ds(0, 1), :], row0_v)

    id_descs = [
        pltpu.async_copy(
            idtab_hbm.at[ids_v.at[pl.ds(j * GSZ, GSZ)]],
            idrows_v.at[pl.ds(j * GSZ, GSZ), :],
            sem_id,
        )
        for j in range(NG_I)
    ]

    sems_g = (sem_g0, sem_g1)
    sems_o = (sem_o0, sem_o1)

    def fire_chunk(c):
        return [
            pltpu.async_copy(
                titab_hbm.at[tok_v.at[pl.ds(c * CH * L + j * GSZ, GSZ)]],
                grows_v.at[c % 2, pl.ds(j * GSZ, GSZ), :],
                sems_g[c % 2],
            )
            for j in range(NG_T)
        ]

    g_descs = [None] * NCHUNK
    o_descs = [None] * NCHUNK
    g_descs[0] = fire_chunk(0)

    row0a = row0_v[0, pl.ds(0, 16)]
    row0b = row0_v[0, pl.ds(16, 16)]
    iota = lax.iota(jnp.int32, 16)

    for c in range(NCHUNK):
        buf = c % 2
        if c + 1 < NCHUNK:
            g_descs[c + 1] = fire_chunk(c + 1)
        for d in g_descs[c]:
            d.wait()
        if c == 0:
            for d in id_descs:
                d.wait()
        if c >= 2:
            o_descs[c - 2].wait()

        def group_body(g, _):
            rowbase = (c * CH + g * 16) * L
            tidx = rowbase + iota * L
            nz = jnp.zeros((16,), jnp.float32)
            for l in range(L):
                t = plsc.load_gather(tok_v, [tidx + l])
                nz = nz + jnp.where(t == 0, 1.0, 0.0)
            denom = jnp.maximum(jnp.float32(L) - nz, 1.0)
            nz_v[pl.ds(g * 16, 16)] = nz
            inv_v[pl.ds(g * 16, 16)] = 1.0 / denom
            return 0

        lax.fori_loop(0, CH // 16, group_body, 0)

        def row_body(r, _):
            tb = r * L
            acc0 = _treesum(grows_v[buf, tb + l, pl.ds(0, 16)] for l in range(L))
            acc1 = _treesum(grows_v[buf, tb + l, pl.ds(16, 16)] for l in range(L))
            nzr = nz_v[r]
            invr = inv_v[r]
            out_v[buf, r, pl.ds(0, 16)] = idrows_v[c * CH + r, pl.ds(0, 16)]
            out_v[buf, r, pl.ds(16, 16)] = idrows_v[c * CH + r, pl.ds(16, 16)]
            out_v[buf, r, pl.ds(32, 16)] = (acc0 - nzr * row0a) * invr
            out_v[buf, r, pl.ds(48, 16)] = (acc1 - nzr * row0b) * invr
            return 0

        lax.fori_loop(0, CH, row_body, 0)

        o_descs[c] = pltpu.async_copy(
            out_v.at[buf],
            out_hbm.at[pl.ds(base + c * CH, CH), :],
            sems_o[buf],
        )

    o_descs[NCHUNK - 2].wait()
    o_descs[NCHUNK - 1].wait()


@jax.jit
def kernel(movie_id, movie_title_tokens, id_embedding_table, title_embedding_table):
    toks_flat = movie_title_tokens.reshape(B * L)
    run = pl.kernel(
        _body,
        out_type=jax.ShapeDtypeStruct((B, DD), jnp.float32),
        mesh=plsc.VectorSubcoreMesh(core_axis_name="c", subcore_axis_name="s"),
        scratch_types=[
            pltpu.VMEM((BPW * L,), jnp.int32),
            pltpu.VMEM((BPW,), jnp.int32),
            pltpu.VMEM((BPW, D), jnp.float32),
            pltpu.VMEM((2, CH * L, D), jnp.float32),
            pltpu.VMEM((2, CH, DD), jnp.float32),
            pltpu.VMEM((1, D), jnp.float32),
            pltpu.VMEM((CH,), jnp.float32),
            pltpu.VMEM((CH,), jnp.float32),
            pltpu.SemaphoreType.DMA,
            pltpu.SemaphoreType.DMA,
            pltpu.SemaphoreType.DMA,
            pltpu.SemaphoreType.DMA,
            pltpu.SemaphoreType.DMA,
        ],
    )
    return run(movie_id, toks_flat, id_embedding_table, title_embedding_table)


# speedup vs baseline: 12.3516x; 12.3516x over previous
import jax
import jax.numpy as jnp
from jax import lax
from jax.experimental import pallas as pl
from jax.experimental.pallas import tpu as pltpu
from jax.experimental.pallas import tpu_sc as plsc

B = 16384
L = 20
D = 32
DD = 2 * D
NC = 2
NS = 16
NW = NC * NS
BPW = B // NW
CH = 64
NCHUNK = BPW // CH
GSZ = 128
NG_T = (CH * L) // GSZ
NG_I = BPW // GSZ


def _treesum(vs):
    vs = list(vs)
    while len(vs) > 1:
        nxt = [vs[i] + vs[i + 1] for i in range(0, len(vs) - 1, 2)]
        if len(vs) % 2:
            nxt.append(vs[-1])
        vs = nxt
    return vs[0]


def _body(ids_hbm, toks_hbm, idtab_hbm, titab_hbm, out_hbm,
          tok_v, ids_v, idrows_v, grows_v, out_v, row0_v, nz_v, inv_v,
          sem_id, sem_g0, sem_g1, sem_o0, sem_o1):
    wid = lax.axis_index("s") * NC + lax.axis_index("c")
    base = wid * BPW

    pltpu.sync_copy(toks_hbm.at[pl.ds(base * L, BPW * L)], tok_v)
    pltpu.sync_copy(ids_hbm.at[pl.ds(base, BPW)], ids_v)
    pltpu.sync_copy(titab_hbm.at[pl.ds(0, 1), :], row0_v)

    id_descs = [
        pltpu.async_copy(
            idtab_hbm.at[ids_v.at[pl.ds(j * GSZ, GSZ)]],
            idrows_v.at[pl.ds(j * GSZ, GSZ), :],
            sem_id,
        )
        for j in range(NG_I)
    ]

    sems_g = (sem_g0, sem_g1)
    sems_o = (sem_o0, sem_o1)

    def fire_chunk(c):
        return [
            pltpu.async_copy(
                titab_hbm.at[tok_v.at[pl.ds(c * CH * L + j * GSZ, GSZ)]],
                grows_v.at[c % 2, pl.ds(j * GSZ, GSZ), :],
                sems_g[c % 2],
            )
            for j in range(NG_T)
        ]

    g_descs = [None] * NCHUNK
    o_descs = [None] * NCHUNK
    g_descs[0] = fire_chunk(0)

    row0a = row0_v[0, pl.ds(0, 16)]
    row0b = row0_v[0, pl.ds(16, 16)]
    iota = lax.iota(jnp.int32, 16)

    for c in range(NCHUNK):
        buf = c % 2
        if c + 1 < NCHUNK:
            g_descs[c + 1] = fire_chunk(c + 1)
        for d in g_descs[c]:
            d.wait()
        if c == 0:
            for d in id_descs:
                d.wait()
        if c >= 2:
            o_descs[c - 2].wait()

        def group_body(g, _):
            rowbase = (c * CH + g * 16) * L
            tidx = rowbase + iota * L
            nz = jnp.zeros((16,), jnp.float32)
            for l in range(L):
                t = plsc.load_gather(tok_v, [tidx + l])
                nz = nz + jnp.where(t == 0, 1.0, 0.0)
            denom = jnp.maximum(jnp.float32(L) - nz, 1.0)
            nz_v[pl.ds(g * 16, 16)] = nz
            inv_v[pl.ds(g * 16, 16)] = 1.0 / denom
            return 0

        lax.fori_loop(0, CH // 16, group_body, 0)

        def row_body(r, _):
            tb = r * L
            acc0 = _treesum(grows_v[buf, tb + l, pl.ds(0, 16)] for l in range(L))
            acc1 = _treesum(grows_v[buf, tb + l, pl.ds(16, 16)] for l in range(L))
            nzr = nz_v[pl.ds(r, 16)][0]
            invr = inv_v[pl.ds(r, 16)][0]
            out_v[buf, r, pl.ds(0, 16)] = idrows_v[c * CH + r, pl.ds(0, 16)]
            out_v[buf, r, pl.ds(16, 16)] = idrows_v[c * CH + r, pl.ds(16, 16)]
            out_v[buf, r, pl.ds(32, 16)] = (acc0 - nzr * row0a) * invr
            out_v[buf, r, pl.ds(48, 16)] = (acc1 - nzr * row0b) * invr
            return 0

        lax.fori_loop(0, CH, row_body, 0)

        o_descs[c] = pltpu.async_copy(
            out_v.at[buf],
            out_hbm.at[pl.ds(base + c * CH, CH), :],
            sems_o[buf],
        )

    o_descs[NCHUNK - 2].wait()
    o_descs[NCHUNK - 1].wait()


@jax.jit
def kernel(movie_id, movie_title_tokens, id_embedding_table, title_embedding_table):
    toks_flat = movie_title_tokens.reshape(B * L)
    run = pl.kernel(
        _body,
        out_type=jax.ShapeDtypeStruct((B, DD), jnp.float32),
        mesh=plsc.VectorSubcoreMesh(core_axis_name="c", subcore_axis_name="s"),
        compiler_params=pltpu.CompilerParams(
            needs_layout_passes=False, use_tc_tiling_on_sc=False
        ),
        scratch_types=[
            pltpu.VMEM((BPW * L,), jnp.int32),
            pltpu.VMEM((BPW,), jnp.int32),
            pltpu.VMEM((BPW, D), jnp.float32),
            pltpu.VMEM((2, CH * L, D), jnp.float32),
            pltpu.VMEM((2, CH, DD), jnp.float32),
            pltpu.VMEM((1, D), jnp.float32),
            pltpu.VMEM((CH + 16,), jnp.float32),
            pltpu.VMEM((CH + 16,), jnp.float32),
            pltpu.SemaphoreType.DMA,
            pltpu.SemaphoreType.DMA,
            pltpu.SemaphoreType.DMA,
            pltpu.SemaphoreType.DMA,
            pltpu.SemaphoreType.DMA,
        ],
    )
    return run(movie_id, toks_flat, id_embedding_table, title_embedding_table)


# speedup vs baseline: 13.7583x; 1.1139x over previous
import jax
import jax.numpy as jnp
from jax import lax
from jax.experimental import pallas as pl
from jax.experimental.pallas import tpu as pltpu
from jax.experimental.pallas import tpu_sc as plsc

B = 16384
L = 20
D = 32
DD = 2 * D
NC = 2
NS = 16
NW = NC * NS
BPW = B // NW
CH = 128
NCHUNK = BPW // CH
GSZ = 128
NG_I = BPW // GSZ


def _body(ids_hbm, toksT_hbm, idtab_hbm, titab_hbm, out_hbm,
          tokT_v, ids_v, idrows_v, sum_v, out_v, row0_v, nz_v, inv_v,
          sem_in, sem_id, sem_g0, sem_g1, sem_o0, sem_o1):
    wid = lax.axis_index("s") * NC + lax.axis_index("c")
    base = wid * BPW

    in_descs = [
        pltpu.async_copy(toksT_hbm.at[l, pl.ds(base, BPW)], tokT_v.at[l], sem_in)
        for l in range(L)
    ]
    pltpu.sync_copy(ids_hbm.at[pl.ds(base, BPW)], ids_v)
    pltpu.sync_copy(titab_hbm.at[pl.ds(0, 1), :], row0_v)
    for d in in_descs:
        d.wait()

    id_descs = [
        pltpu.async_copy(
            idtab_hbm.at[ids_v.at[pl.ds(j * GSZ, GSZ)]],
            idrows_v.at[pl.ds(j * GSZ, GSZ), :],
            sem_id,
        )
        for j in range(NG_I)
    ]

    sems_g = (sem_g0, sem_g1)
    sems_o = (sem_o0, sem_o1)
    zero16 = jnp.zeros((16,), jnp.float32)
    g_descs = [None] * NCHUNK
    o_descs = [None] * NCHUNK

    row0a = row0_v[0, pl.ds(0, 16)]
    row0b = row0_v[0, pl.ds(16, 16)]

    def prep_chunk(c):
        buf = c % 2

        def zero_body(r, _):
            sum_v[buf, r, pl.ds(0, 16)] = zero16
            sum_v[buf, r, pl.ds(16, 16)] = zero16
            return 0

        lax.fori_loop(0, CH, zero_body, 0)

        def group_body(g, _):
            rs = c * CH + g * 16
            nz = jnp.zeros((16,), jnp.float32)
            for l in range(L):
                t = tokT_v[l, pl.ds(rs, 16)]
                nz = nz + jnp.where(t == 0, 1.0, 0.0)
            denom = jnp.maximum(jnp.float32(L) - nz, 1.0)
            nz_v[buf, pl.ds(g * 16, 16)] = nz
            inv_v[buf, pl.ds(g * 16, 16)] = 1.0 / denom
            return 0

        lax.fori_loop(0, CH // 16, group_body, 0)

        return [
            pltpu.async_copy(
                titab_hbm.at[tokT_v.at[l, pl.ds(c * CH, CH)]],
                sum_v.at[buf],
                sems_g[buf],
                add=True,
            )
            for l in range(L)
        ]

    g_descs[0] = prep_chunk(0)

    for c in range(NCHUNK):
        buf = c % 2
        if c + 1 < NCHUNK:
            g_descs[c + 1] = prep_chunk(c + 1)
        for d in g_descs[c]:
            d.wait()
        if c == 0:
            for d in id_descs:
                d.wait()
        if c >= 2:
            o_descs[c - 2].wait()

        def row_body(r, _):
            s0 = sum_v[buf, r, pl.ds(0, 16)]
            s1 = sum_v[buf, r, pl.ds(16, 16)]
            nzr = nz_v[buf, pl.ds(r, 16)][0]
            invr = inv_v[buf, pl.ds(r, 16)][0]
            out_v[buf, r, pl.ds(0, 16)] = idrows_v[c * CH + r, pl.ds(0, 16)]
            out_v[buf, r, pl.ds(16, 16)] = idrows_v[c * CH + r, pl.ds(16, 16)]
            out_v[buf, r, pl.ds(32, 16)] = (s0 - nzr * row0a) * invr
            out_v[buf, r, pl.ds(48, 16)] = (s1 - nzr * row0b) * invr
            return 0

        lax.fori_loop(0, CH, row_body, 0)

        o_descs[c] = pltpu.async_copy(
            out_v.at[buf],
            out_hbm.at[pl.ds(base + c * CH, CH), :],
            sems_o[buf],
        )

    o_descs[NCHUNK - 2].wait()
    o_descs[NCHUNK - 1].wait()


@jax.jit
def kernel(movie_id, movie_title_tokens, id_embedding_table, title_embedding_table):
    toksT = movie_title_tokens.T
    run = pl.kernel(
        _body,
        out_type=jax.ShapeDtypeStruct((B, DD), jnp.float32),
        mesh=plsc.VectorSubcoreMesh(core_axis_name="c", subcore_axis_name="s"),
        compiler_params=pltpu.CompilerParams(
            needs_layout_passes=False, use_tc_tiling_on_sc=False
        ),
        scratch_types=[
            pltpu.VMEM((L, BPW), jnp.int32),
            pltpu.VMEM((BPW,), jnp.int32),
            pltpu.VMEM((BPW, D), jnp.float32),
            pltpu.VMEM((2, CH, D), jnp.float32),
            pltpu.VMEM((2, CH, DD), jnp.float32),
            pltpu.VMEM((1, D), jnp.float32),
            pltpu.VMEM((2, CH + 16), jnp.float32),
            pltpu.VMEM((2, CH + 16), jnp.float32),
            pltpu.SemaphoreType.DMA,
            pltpu.SemaphoreType.DMA,
            pltpu.SemaphoreType.DMA,
            pltpu.SemaphoreType.DMA,
            pltpu.SemaphoreType.DMA,
            pltpu.SemaphoreType.DMA,
        ],
    )
    return run(movie_id, toksT, id_embedding_table, title_embedding_table)
